# baseline (speedup 1.0000x reference)
import jax
import jax.numpy as jnp
from jax.experimental import pallas as pl
from jax.experimental.pallas import tpu as pltpu

BN = 16384


def _tc_body(tab_ref, b_ref, w_ref, x_ref, o_ref):
    t0, t1, t2 = tab_ref[0], tab_ref[1], tab_ref[2]
    fields = x_ref.shape[0]
    w = w_ref[...].reshape(fields, 1)
    wt0 = w * t0
    wt1 = w * t1
    wt2 = w * t2
    x = x_ref[...]
    y = jnp.where(x == 0, wt0, jnp.where(x == 1, wt1, wt2))
    o_ref[...] = (jnp.sum(y, axis=0) + b_ref[0]).reshape(1, BN)


def kernel(x, table, W, b):
    batch, fields = x.shape
    grid = (batch // BN,)
    out = pl.pallas_call(
        _tc_body,
        grid=grid,
        in_specs=[
            pl.BlockSpec(memory_space=pltpu.SMEM),
            pl.BlockSpec(memory_space=pltpu.SMEM),
            pl.BlockSpec((fields,), lambda i: (0,)),
            pl.BlockSpec((fields, BN), lambda i: (0, i)),
        ],
        out_specs=pl.BlockSpec((1, BN), lambda i: (0, i)),
        out_shape=jax.ShapeDtypeStruct((1, batch), jnp.float32),
        compiler_params=pltpu.CompilerParams(
            dimension_semantics=("parallel",),
        ),
    )(table, b, W.reshape(fields), x.T)
    return out.reshape(batch, 1)


# speedup vs baseline: 1.1459x; 1.1459x over previous
import jax
import jax.numpy as jnp
from jax.experimental import pallas as pl
from jax.experimental.pallas import tpu as pltpu

BN = 8192


def _tc_body(tab_ref, b_ref, w_ref, x_ref, o_ref):
    t0, t1, t2 = tab_ref[0], tab_ref[1], tab_ref[2]
    fields = x_ref.shape[0]
    w = w_ref[...].reshape(fields, 1)
    wt0 = w * t0
    wt1 = w * t1
    wt2 = w * t2
    x = x_ref[...]
    y = jnp.where(x == 0, wt0, jnp.where(x == 1, wt1, wt2))
    o_ref[...] = (jnp.sum(y, axis=0) + b_ref[0]).reshape(1, BN)


def kernel(x, table, W, b):
    batch, fields = x.shape
    grid = (batch // BN,)
    out = pl.pallas_call(
        _tc_body,
        grid=grid,
        in_specs=[
            pl.BlockSpec(memory_space=pltpu.SMEM),
            pl.BlockSpec(memory_space=pltpu.SMEM),
            pl.BlockSpec((fields,), lambda i: (0,)),
            pl.BlockSpec((fields, BN), lambda i: (0, i)),
        ],
        out_specs=pl.BlockSpec((1, BN), lambda i: (0, i)),
        out_shape=jax.ShapeDtypeStruct((1, batch), jnp.float32),
        compiler_params=pltpu.CompilerParams(
            dimension_semantics=("parallel",),
        ),
    )(table, b, W.reshape(fields), x.T)
    return out.reshape(batch, 1)
